# baseline (speedup 1.0000x reference)
import functools

import jax
import jax.numpy as jnp
from jax import lax
from jax.experimental import pallas as pl
from jax.experimental.pallas import tpu as pltpu
from jax.experimental.pallas import tpu_sc as plsc

B = 16384
F = 64

_info = plsc.get_sparse_core_info()
NC = _info.num_cores
NS = _info.num_subcores
L = _info.num_lanes
NW = NC * NS
BPW = B // NW
CH = 128
NCH = BPW // CH

_mesh = plsc.VectorSubcoreMesh(core_axis_name="c", subcore_axis_name="s")


@functools.partial(
    pl.kernel,
    mesh=_mesh,
    compiler_params=pltpu.CompilerParams(
        needs_layout_passes=False, use_tc_tiling_on_sc=False),
    out_type=jax.ShapeDtypeStruct((B,), jnp.float32),
    scratch_types=[
        pltpu.VMEM((NCH, CH), jnp.int32),
        pltpu.VMEM((NCH, CH), jnp.int32),
        pltpu.VMEM((BPW, F), jnp.float32),
        pltpu.VMEM((BPW, F), jnp.float32),
        pltpu.VMEM((BPW,), jnp.float32),
        pltpu.SemaphoreType.DMA,
    ],
)
def _mf_sc(user_hbm, item_hbm, uf_hbm, if_hbm, out_hbm,
           uidx, iidx, urows, irows, outv, sem):
    wid = lax.axis_index("s") * NC + lax.axis_index("c")

    pltpu.sync_copy(user_hbm.at[wid], uidx)
    pltpu.sync_copy(item_hbm.at[wid], iidx)

    copies = []
    for k in range(NCH):
        copies.append(pltpu.async_copy(
            uf_hbm.at[uidx.at[k]], urows.at[pl.ds(k * CH, CH)], sem))
        copies.append(pltpu.async_copy(
            if_hbm.at[iidx.at[k]], irows.at[pl.ds(k * CH, CH)], sem))
    for c in copies:
        c.wait()

    lanes = lax.iota(jnp.int32, L)

    def group_body(g, _):
        acc = jnp.zeros((L,), jnp.float32)
        for rr in range(L):
            r = g * L + rr
            p = jnp.zeros((L,), jnp.float32)
            for j in range(F // L):
                u = urows[r, pl.ds(j * L, L)]
                v = irows[r, pl.ds(j * L, L)]
                p = p + u * v
            acc = jnp.where(lanes == rr, jnp.sum(p), acc)
        outv[pl.ds(g * L, L)] = acc
        return 0

    lax.fori_loop(0, BPW // L, group_body, 0)

    pltpu.sync_copy(outv, out_hbm.at[pl.ds(wid * BPW, BPW)])


def kernel(user, item, user_factors, item_factors):
    user_r = user.astype(jnp.int32).reshape(NW, NCH, CH)
    item_r = item.astype(jnp.int32).reshape(NW, NCH, CH)
    return _mf_sc(user_r, item_r, user_factors, item_factors)
